# baseline (speedup 1.0000x reference)
import functools

import jax
import jax.numpy as jnp
from jax import lax
from jax.experimental import pallas as pl
from jax.experimental.pallas import tpu as pltpu
from jax.experimental.pallas import tpu_sc as plsc

N_TIME = 16
N_SPACE = 99
_TPAD = 128
_NS = 16
_L = 16


def _table_body(obs_ref, alpha_ref, mu_ref, beta_ref, out_ref):
    beta = beta_ref[0, 0]
    obs_f = obs_ref[...].astype(jnp.float32)
    g = lax.dot_general(obs_f, alpha_ref[...], (((1,), (0,)), ((), ())),
                        precision=lax.Precision.HIGHEST,
                        preferred_element_type=jnp.float32)
    tq = lax.broadcasted_iota(jnp.int32, (N_TIME, N_TIME), 0)
    tp = lax.broadcasted_iota(jnp.int32, (N_TIME, N_TIME), 1)
    dt = (tq - tp).astype(jnp.float32)
    w = jnp.where(tp < tq, beta * jnp.exp(-beta * dt), 0.0)
    h = lax.dot_general(w, g, (((1,), (0,)), ((), ())),
                        precision=lax.Precision.HIGHEST,
                        preferred_element_type=jnp.float32)
    out_ref[:, pl.ds(0, N_SPACE)] = jnp.maximum(mu_ref[...] + h, 0.0)


def _build_table(obs, mu, alpha, beta):
    return pl.pallas_call(
        _table_body,
        out_shape=jax.ShapeDtypeStruct((N_TIME, _TPAD), jnp.float32),
    )(obs, alpha, mu.reshape(1, N_SPACE), beta.reshape(1, 1))


def _gather_sc(table, t, s):
    batch = t.shape[0]
    bpw = batch // _NS
    mesh = plsc.VectorSubcoreMesh(core_axis_name="c", subcore_axis_name="s",
                                  num_cores=1)

    @functools.partial(
        pl.kernel,
        out_type=jax.ShapeDtypeStruct((batch,), jnp.float32),
        mesh=mesh,
        compiler_params=pltpu.CompilerParams(needs_layout_passes=False,
                                            skip_device_barrier=True),
        scratch_types=[
            pltpu.VMEM((N_TIME, _TPAD), jnp.float32),
            pltpu.VMEM((bpw,), jnp.int32),
            pltpu.VMEM((bpw,), jnp.int32),
            pltpu.VMEM((bpw,), jnp.float32),
            pltpu.SemaphoreType.DMA,
        ],
    )
    def gather_kernel(table_hbm, t_hbm, s_hbm, out_hbm,
                      table_v, t_v, s_v, out_v, sem):
        sid = lax.axis_index("s")
        base = sid * bpw
        copies = [
            pltpu.async_copy(table_hbm, table_v, sem),
            pltpu.async_copy(t_hbm.at[pl.ds(base, bpw)], t_v, sem),
            pltpu.async_copy(s_hbm.at[pl.ds(base, bpw)], s_v, sem),
        ]
        for c in copies:
            c.wait()
        for j in range(bpw // _L):
            tv = t_v[pl.ds(j * _L, _L)]
            sv = s_v[pl.ds(j * _L, _L)]
            out_v[pl.ds(j * _L, _L)] = plsc.load_gather(table_v, [tv, sv])
        pltpu.sync_copy(out_v, out_hbm.at[pl.ds(base, bpw)])

    return gather_kernel(table, t, s)


def kernel(t, s, obs, mu, alpha, beta):
    table = _build_table(obs, mu, alpha, beta)
    return _gather_sc(table, t.astype(jnp.int32), s.astype(jnp.int32))


# speedup vs baseline: 1.0001x; 1.0001x over previous
import functools

import jax
import jax.numpy as jnp
from jax import lax
from jax.experimental import pallas as pl
from jax.experimental.pallas import tpu as pltpu
from jax.experimental.pallas import tpu_sc as plsc

N_TIME = 16
N_SPACE = 99
_TPAD = 128
_NS = 8
_L = 16


def _table_body(obs_ref, alpha_ref, mu_ref, beta_ref, out_ref):
    beta = beta_ref[0, 0]
    obs_f = obs_ref[...].astype(jnp.float32)
    g = lax.dot_general(obs_f, alpha_ref[...], (((1,), (0,)), ((), ())),
                        precision=lax.Precision.HIGHEST,
                        preferred_element_type=jnp.float32)
    tq = lax.broadcasted_iota(jnp.int32, (N_TIME, N_TIME), 0)
    tp = lax.broadcasted_iota(jnp.int32, (N_TIME, N_TIME), 1)
    dt = (tq - tp).astype(jnp.float32)
    w = jnp.where(tp < tq, beta * jnp.exp(-beta * dt), 0.0)
    h = lax.dot_general(w, g, (((1,), (0,)), ((), ())),
                        precision=lax.Precision.HIGHEST,
                        preferred_element_type=jnp.float32)
    out_ref[:, pl.ds(0, N_SPACE)] = jnp.maximum(mu_ref[...] + h, 0.0)


def _build_table(obs, mu, alpha, beta):
    return pl.pallas_call(
        _table_body,
        out_shape=jax.ShapeDtypeStruct((N_TIME, _TPAD), jnp.float32),
    )(obs, alpha, mu.reshape(1, N_SPACE), beta.reshape(1, 1))


def _gather_sc(table, t, s):
    batch = t.shape[0]
    bpw = batch // _NS
    mesh = plsc.VectorSubcoreMesh(core_axis_name="c", subcore_axis_name="s",
                                  num_cores=1, num_subcores=8)

    @functools.partial(
        pl.kernel,
        out_type=jax.ShapeDtypeStruct((batch,), jnp.float32),
        mesh=mesh,
        compiler_params=pltpu.CompilerParams(needs_layout_passes=False),
        scratch_types=[
            pltpu.VMEM((N_TIME, _TPAD), jnp.float32),
            pltpu.VMEM((bpw,), jnp.int32),
            pltpu.VMEM((bpw,), jnp.int32),
            pltpu.VMEM((bpw,), jnp.float32),
            pltpu.SemaphoreType.DMA,
        ],
    )
    def gather_kernel(table_hbm, t_hbm, s_hbm, out_hbm,
                      table_v, t_v, s_v, out_v, sem):
        sid = lax.axis_index("s")
        base = sid * bpw
        copies = [
            pltpu.async_copy(table_hbm, table_v, sem),
            pltpu.async_copy(t_hbm.at[pl.ds(base, bpw)], t_v, sem),
            pltpu.async_copy(s_hbm.at[pl.ds(base, bpw)], s_v, sem),
        ]
        for c in copies:
            c.wait()
        for j in range(bpw // _L):
            tv = t_v[pl.ds(j * _L, _L)]
            sv = s_v[pl.ds(j * _L, _L)]
            out_v[pl.ds(j * _L, _L)] = plsc.load_gather(table_v, [tv, sv])
        pltpu.sync_copy(out_v, out_hbm.at[pl.ds(base, bpw)])

    return gather_kernel(table, t, s)


def kernel(t, s, obs, mu, alpha, beta):
    table = _build_table(obs, mu, alpha, beta)
    return _gather_sc(table, t.astype(jnp.int32), s.astype(jnp.int32))


# speedup vs baseline: 1.0026x; 1.0025x over previous
import functools

import jax
import jax.numpy as jnp
from jax import lax
from jax.experimental import pallas as pl
from jax.experimental.pallas import tpu as pltpu
from jax.experimental.pallas import tpu_sc as plsc

N_TIME = 16
N_SPACE = 99
_TPAD = 128
_NS = 16
_L = 16


def _table_body(obs_ref, alpha_ref, mu_ref, beta_ref, out_ref):
    beta = beta_ref[0, 0]
    obs_f = obs_ref[...].astype(jnp.float32)
    g = lax.dot_general(obs_f, alpha_ref[...], (((1,), (0,)), ((), ())),
                        precision=lax.Precision.HIGHEST,
                        preferred_element_type=jnp.float32)
    tq = lax.broadcasted_iota(jnp.int32, (N_TIME, N_TIME), 0)
    tp = lax.broadcasted_iota(jnp.int32, (N_TIME, N_TIME), 1)
    dt = (tq - tp).astype(jnp.float32)
    w = jnp.where(tp < tq, beta * jnp.exp(-beta * dt), 0.0)
    h = lax.dot_general(w, g, (((1,), (0,)), ((), ())),
                        precision=lax.Precision.HIGHEST,
                        preferred_element_type=jnp.float32)
    out_ref[:, pl.ds(0, N_SPACE)] = jnp.maximum(mu_ref[...] + h, 0.0)


def _build_table(obs, mu, alpha, beta):
    return pl.pallas_call(
        _table_body,
        out_shape=jax.ShapeDtypeStruct((N_TIME, _TPAD), jnp.float32),
    )(obs, alpha, mu.reshape(1, N_SPACE), beta.reshape(1, 1))


def _gather_sc(table, t, s):
    batch = t.shape[0]
    bpw = batch // _NS
    mesh = plsc.VectorSubcoreMesh(core_axis_name="c", subcore_axis_name="s",
                                  num_cores=1)

    @functools.partial(
        pl.kernel,
        out_type=jax.ShapeDtypeStruct((batch,), jnp.float32),
        mesh=mesh,
        compiler_params=pltpu.CompilerParams(needs_layout_passes=False),
        scratch_types=[
            pltpu.VMEM((N_TIME, _TPAD), jnp.float32),
            pltpu.VMEM((bpw,), jnp.int32),
            pltpu.VMEM((bpw,), jnp.int32),
            pltpu.VMEM((bpw,), jnp.float32),
            pltpu.SemaphoreType.DMA,
        ],
    )
    def gather_kernel(table_hbm, t_hbm, s_hbm, out_hbm,
                      table_v, t_v, s_v, out_v, sem):
        sid = lax.axis_index("s")
        base = sid * bpw
        copies = [
            pltpu.async_copy(table_hbm, table_v, sem),
            pltpu.async_copy(t_hbm.at[pl.ds(base, bpw)], t_v, sem),
            pltpu.async_copy(s_hbm.at[pl.ds(base, bpw)], s_v, sem),
        ]
        for c in copies:
            c.wait()
        for j in range(bpw // _L):
            tv = t_v[pl.ds(j * _L, _L)]
            sv = s_v[pl.ds(j * _L, _L)]
            out_v[pl.ds(j * _L, _L)] = plsc.load_gather(table_v, [tv, sv])
        pltpu.sync_copy(out_v, out_hbm.at[pl.ds(base, bpw)])

    return gather_kernel(table, t, s)


def kernel(t, s, obs, mu, alpha, beta):
    table = _build_table(obs, mu, alpha, beta)
    return _gather_sc(table, t.astype(jnp.int32), s.astype(jnp.int32))
